# Initial kernel scaffold; baseline (speedup 1.0000x reference)
import jax
import jax.numpy as jnp
from jax.experimental import pallas as pl


def kernel(n, edge_index, e, W_self1, W_neigh1, b1, W_self2, W_neigh2, b2, W_aw, b_aw, W_t1, b_t1, W_t2, b_t2):
    raise NotImplementedError("write your pallas kernel here")


# speedup vs baseline: 3.7410x; 3.7410x over previous
import functools

import jax
import jax.numpy as jnp
from jax import lax
from jax.experimental import pallas as pl
from jax.experimental.pallas import tpu as pltpu
from jax.experimental.pallas import tpu_sc as plsc

N = 10000
D = 128
NC = 2
NS = 16
NW = NC * NS
E = 320000
EP = E // NW
CHUNK = 128
CHUNKS = 80
EPP = CHUNK * CHUNKS
NPAD = 10240
RPT = NPAD // NS
ROWBUF = 128


def _make_edge_scatter(with_deg):
  mesh = plsc.VectorSubcoreMesh(core_axis_name="c", subcore_axis_name="s")

  out_type = [jax.ShapeDtypeStruct((NC * NPAD, D), jnp.float32)]
  scratch = [
      pltpu.VMEM((CHUNKS, CHUNK), jnp.int32),
      pltpu.VMEM((CHUNKS, CHUNK), jnp.int32),
      pltpu.VMEM((CHUNK, D), jnp.float32),
      pltpu.VMEM_SHARED((NPAD, D), jnp.float32),
      pltpu.SemaphoreType.DMA,
  ]
  if with_deg:
    out_type.append(jax.ShapeDtypeStruct((NC * NPAD,), jnp.float32))
    scratch += [
        pltpu.VMEM_SHARED((NPAD,), jnp.float32),
        pltpu.VMEM((RPT,), jnp.float32),
        pltpu.VMEM((CHUNK,), jnp.float32),
    ]

  @functools.partial(pl.kernel, out_type=out_type, mesh=mesh,
                     scratch_types=scratch)
  def edge_scatter(table, srcp, dstp, *rest):
    if with_deg:
      out, deg_out, src_v, dst_v, rows_v, acc, sem, deg_sp, zbuf, ones_v = rest
    else:
      out, src_v, dst_v, rows_v, acc, sem = rest
    cid = lax.axis_index("c")
    sid = lax.axis_index("s")
    wid = sid * NC + cid

    z16 = jnp.zeros((16,), jnp.float32)

    def zrow(i, carry):
      for k in range(D // 16):
        rows_v[i, pl.ds(k * 16, 16)] = z16
      return carry

    lax.fori_loop(0, CHUNK, zrow, 0)
    for b in range(RPT // ROWBUF):
      pltpu.sync_copy(rows_v, acc.at[pl.ds(sid * RPT + b * ROWBUF, ROWBUF)])

    if with_deg:
      o16 = jnp.ones((16,), jnp.float32)

      def zdeg(i, carry):
        zbuf[pl.ds(i * 16, 16)] = z16
        return carry

      lax.fori_loop(0, RPT // 16, zdeg, 0)
      for k in range(CHUNK // 16):
        ones_v[pl.ds(k * 16, 16)] = o16
      pltpu.sync_copy(zbuf, deg_sp.at[pl.ds(sid * RPT, RPT)])

    pltpu.sync_copy(srcp.at[pl.ds(wid * CHUNKS, CHUNKS)], src_v)
    pltpu.sync_copy(dstp.at[pl.ds(wid * CHUNKS, CHUNKS)], dst_v)

    plsc.subcore_barrier()

    def body(j, carry):
      pltpu.async_copy(table.at[src_v.at[j]], rows_v, sem).wait()
      pltpu.sync_copy(rows_v, acc.at[dst_v.at[j]], add=True)
      if with_deg:
        pltpu.sync_copy(ones_v, deg_sp.at[dst_v.at[j]], add=True)
      return carry

    lax.fori_loop(0, CHUNKS, body, 0)

    plsc.subcore_barrier()

    pltpu.sync_copy(acc.at[pl.ds(sid * RPT, RPT)],
                    out.at[pl.ds(cid * NPAD + sid * RPT, RPT)])
    if with_deg:
      pltpu.sync_copy(deg_sp.at[pl.ds(sid * RPT, RPT)],
                      deg_out.at[pl.ds(cid * NPAD + sid * RPT, RPT)])

  return edge_scatter


_edge_scatter_deg = _make_edge_scatter(True)
_edge_scatter = _make_edge_scatter(False)


BN = 512
GRID = NPAD // BN


def _dense1_body(x_ref, wn_ref, ws_ref, b_ref, y_ref, s_ref):
  x = x_ref[...]
  y_ref[...] = jnp.dot(x, wn_ref[...], preferred_element_type=jnp.float32)
  s_ref[...] = (jnp.dot(x, ws_ref[...], preferred_element_type=jnp.float32)
                + b_ref[...])


def _dense1(x, wn, ws, b):
  return pl.pallas_call(
      _dense1_body,
      grid=(GRID,),
      in_specs=[
          pl.BlockSpec((BN, D), lambda i: (i, 0)),
          pl.BlockSpec((D, D), lambda i: (0, 0)),
          pl.BlockSpec((D, D), lambda i: (0, 0)),
          pl.BlockSpec((1, D), lambda i: (0, 0)),
      ],
      out_specs=[
          pl.BlockSpec((BN, D), lambda i: (i, 0)),
          pl.BlockSpec((BN, D), lambda i: (i, 0)),
      ],
      out_shape=[
          jax.ShapeDtypeStruct((NPAD, D), jnp.float32),
          jax.ShapeDtypeStruct((NPAD, D), jnp.float32),
      ],
  )(x, wn, ws, b)


def _dense2_body(pf_ref, pd_ref, s1_ref, wn_ref, ws_ref, b_ref, y_ref, s_ref):
  agg = pf_ref[0] + pf_ref[1]
  deg = jnp.maximum(pd_ref[0] + pd_ref[1], 1.0)
  h = jax.nn.sigmoid(s1_ref[...] + agg / deg)
  y_ref[...] = jnp.dot(h, wn_ref[...], preferred_element_type=jnp.float32)
  s_ref[...] = (jnp.dot(h, ws_ref[...], preferred_element_type=jnp.float32)
                + b_ref[...])


def _dense2(pf, pd, s1, wn, ws, b):
  return pl.pallas_call(
      _dense2_body,
      grid=(GRID,),
      in_specs=[
          pl.BlockSpec((NC, BN, D), lambda i: (0, i, 0)),
          pl.BlockSpec((NC, BN, 1), lambda i: (0, i, 0)),
          pl.BlockSpec((BN, D), lambda i: (i, 0)),
          pl.BlockSpec((D, D), lambda i: (0, 0)),
          pl.BlockSpec((D, D), lambda i: (0, 0)),
          pl.BlockSpec((1, D), lambda i: (0, 0)),
      ],
      out_specs=[
          pl.BlockSpec((BN, D), lambda i: (i, 0)),
          pl.BlockSpec((BN, D), lambda i: (i, 0)),
      ],
      out_shape=[
          jax.ShapeDtypeStruct((NPAD, D), jnp.float32),
          jax.ShapeDtypeStruct((NPAD, D), jnp.float32),
      ],
  )(pf, pd, s1, wn, ws, b)


def _readout_body(pf_ref, pd_ref, s2_ref, waw_ref, baw_ref,
                  wt1a_ref, wt1b_ref, bt1_ref, wt2_ref, bt2_ref, out_ref):
  agg = pf_ref[0] + pf_ref[1]
  deg = jnp.maximum(pd_ref[0] + pd_ref[1], 1.0)
  h = jax.nn.sigmoid(s2_ref[...] + agg / deg)
  valid = lax.broadcasted_iota(jnp.int32, (NPAD, 1), 0) < N
  wlog = jnp.sum(h * waw_ref[...], axis=1, keepdims=True) + baw_ref[0, 0]
  w = jnp.where(valid, jax.nn.sigmoid(wlog), 0.0)
  h_sum = jnp.sum(w * h, axis=0, keepdims=True)
  h_masked = jnp.where(valid, h, -jnp.inf)
  h_max = jnp.max(h_masked, axis=0, keepdims=True)
  t1 = jax.nn.sigmoid(
      jnp.dot(h_sum, wt1a_ref[...], preferred_element_type=jnp.float32)
      + jnp.dot(h_max, wt1b_ref[...], preferred_element_type=jnp.float32)
      + bt1_ref[...])
  out_ref[...] = jax.nn.sigmoid(
      jnp.sum(t1 * wt2_ref[...], axis=1, keepdims=True) + bt2_ref[...])


def _readout(pf, pd, s2, waw, baw, wt1a, wt1b, bt1, wt2, bt2):
  return pl.pallas_call(
      _readout_body,
      out_shape=jax.ShapeDtypeStruct((1, 1), jnp.float32),
  )(pf, pd, s2, waw, baw, wt1a, wt1b, bt1, wt2, bt2)


def kernel(n, edge_index, e, W_self1, W_neigh1, b1, W_self2, W_neigh2, b2,
           W_aw, b_aw, W_t1, b_t1, W_t2, b_t2):
  del e

  src = edge_index[0]
  dst = edge_index[1]
  pad = EPP - EP
  srcp = jnp.concatenate(
      [src.reshape(NW, EP), jnp.zeros((NW, pad), jnp.int32)], axis=1
  ).reshape(NW * CHUNKS, CHUNK)
  dstp = jnp.concatenate(
      [dst.reshape(NW, EP), jnp.full((NW, pad), N, jnp.int32)], axis=1
  ).reshape(NW * CHUNKS, CHUNK)

  n_pad = jnp.concatenate(
      [n, jnp.zeros((NPAD - N, D), jnp.float32)], axis=0)

  y1, s1 = _dense1(n_pad, W_neigh1, W_self1, b1.reshape(1, D))

  p1, deg = _edge_scatter_deg(y1, srcp, dstp)
  pf1 = p1.reshape(NC, NPAD, D)
  pd = deg.reshape(NC, NPAD, 1)

  y2, s2 = _dense2(pf1, pd, s1, W_neigh2, W_self2, b2.reshape(1, D))

  p2 = _edge_scatter(y2, srcp, dstp)[0]
  pf2 = p2.reshape(NC, NPAD, D)

  return _readout(pf2, pd, s2,
                  W_aw.T, b_aw.reshape(1, 1),
                  W_t1[:D], W_t1[D:], b_t1.reshape(1, D),
                  W_t2.T, b_t2.reshape(1, 1))
